# baseline (speedup 1.0000x reference)
import functools

import jax
import jax.numpy as jnp
from jax import lax
from jax.experimental import pallas as pl
from jax.experimental.pallas import tpu as pltpu
from jax.experimental.pallas import tpu_sc as plsc

_NB = 16
_BS = 64
_HID = 128
_NACT = 16
_FEAT = 3
_ROWS_PER_TILE = 16
_NTILES = _BS // _ROWS_PER_TILE


def _dense_kernel(x_ref, wt_ref, be_ref, wg_ref, bg_ref, w1_ref, b1_ref,
                  w2_ref, b2_ref, out_ref):
    xm = jnp.dot(x_ref[...], wt_ref[...],
                 preferred_element_type=jnp.float32) * (1.0 / _NB) + be_ref[...]
    g = jnp.dot(xm, wg_ref[...], preferred_element_type=jnp.float32) + bg_ref[...]
    h = jnp.dot(g, w1_ref[...], preferred_element_type=jnp.float32) + b1_ref[...]
    h = jnp.where(h >= 0, h, 0.01 * h)
    out_ref[...] = jnp.dot(h, w2_ref[...],
                           preferred_element_type=jnp.float32) + b2_ref[...]


@functools.partial(
    pl.kernel,
    out_type=jax.ShapeDtypeStruct((_BS,), jnp.float32),
    mesh=plsc.VectorSubcoreMesh(core_axis_name="c", subcore_axis_name="s"),
    scratch_types=[
        pltpu.VMEM((_ROWS_PER_TILE, _NACT), jnp.float32),
        pltpu.VMEM((_ROWS_PER_TILE, _NACT), jnp.float32),
        pltpu.VMEM((_ROWS_PER_TILE,), jnp.float32),
    ],
)
def _sc_select(act_hbm, q_hbm, out_hbm, act_v, q_v, out_v):
    wid = lax.axis_index("s") * 2 + lax.axis_index("c")

    @pl.when(wid < _NTILES)
    def _():
        base = wid * _ROWS_PER_TILE
        pltpu.sync_copy(act_hbm.at[pl.ds(base, _ROWS_PER_TILE), :], act_v)
        pltpu.sync_copy(q_hbm.at[pl.ds(base, _ROWS_PER_TILE), :], q_v)
        iota = lax.iota(jnp.int32, _NACT)

        def splat_reduce(v, op):
            for s in (8, 4, 2, 1):
                perm = jnp.bitwise_xor(iota, s)
                v = op(v, v.at[perm].get(mode="promise_in_bounds"))
            return v

        acc = jnp.zeros((_NACT,), jnp.float32)
        for r in range(_ROWS_PER_TILE):
            av = act_v[r, :]
            m_sp = splat_reduce(av, jnp.maximum)
            idx_sp = splat_reduce(jnp.where(av == m_sp, iota, _NACT),
                                  jnp.minimum)
            q_sp = q_v[r, :].at[idx_sp].get(mode="promise_in_bounds")
            acc = jnp.where(iota == r, q_sp, acc)
        out_v[...] = acc
        pltpu.sync_copy(out_v, out_hbm.at[pl.ds(base, _ROWS_PER_TILE)])


def kernel(inps, unary_tensor, W_emb, b_emb, W_gcn, b_gcn, W1, b1, W2, b2):
    actions = inps[0, 1]
    x = unary_tensor.reshape(_BS, _NB * _FEAT)
    wt = jnp.tile(W_emb.T, (_NB, 1))
    all_q = pl.pallas_call(
        _dense_kernel,
        out_shape=jax.ShapeDtypeStruct((_BS, _NACT), jnp.float32),
    )(x, wt, b_emb.reshape(1, _HID), W_gcn.T, b_gcn.reshape(1, _HID),
      W1.T, b1.reshape(1, _HID), W2.T, b2.reshape(1, _NACT))
    return _sc_select(actions, all_q).reshape(_BS, 1)
